# Initial kernel scaffold; baseline (speedup 1.0000x reference)
import jax
import jax.numpy as jnp
from jax.experimental import pallas as pl


def kernel(X_B, batch_A, c_indices, codebook, ema_grad, W, b, warm_up_rate, unlabeled):
    raise NotImplementedError("write your pallas kernel here")


# speedup vs baseline: 1.8065x; 1.8065x over previous
import functools

import jax
import jax.numpy as jnp
from jax import lax
from jax.experimental import pallas as pl
from jax.experimental.pallas import tpu as pltpu
from jax.experimental.pallas import tpu_sc as plsc

B = 16384
D = 256
M = 1024

_info = plsc.get_sparse_core_info()
NC, NS, L = _info.num_cores, _info.num_subcores, _info.num_lanes
NW = NC * NS
RPW = B // NW
CHUNK = 128
NCHUNK = RPW // CHUNK

_sc_mesh = plsc.VectorSubcoreMesh(core_axis_name="c", subcore_axis_name="s")


@functools.partial(
    pl.kernel,
    mesh=_sc_mesh,
    out_type=[
        jax.ShapeDtypeStruct((B, D), jnp.float32),
        jax.ShapeDtypeStruct((B // CHUNK, CHUNK), jnp.int32),
    ],
    scratch_types=[
        pltpu.VMEM((NCHUNK, CHUNK), jnp.int32),
        [pltpu.VMEM((CHUNK,), jnp.int32)] * NCHUNK,
        [pltpu.VMEM((CHUNK, D), jnp.float32)] * 2,
        [pltpu.SemaphoreType.DMA] * 2,
    ],
)
def _sc_gather(src_hbm, cind_hbm, cb_hbm, cg_hbm, sc_hbm,
               src_v, sc_v, row_v, sem):
    c = lax.axis_index("c")
    s = lax.axis_index("s")
    wid = s * NC + c
    base = wid * RPW

    pltpu.sync_copy(src_hbm.at[pl.ds(wid * NCHUNK, NCHUNK), :], src_v)
    for k in range(NCHUNK):
        pltpu.async_copy(cind_hbm.at[src_v.at[k]], sc_v[k], sem[0]).wait()
    for k in range(NCHUNK):
        pltpu.sync_copy(sc_v[k], sc_hbm.at[wid * NCHUNK + k])

    gets = [None, None]
    puts = [None, None]
    for k in range(NCHUNK):
        kb = k % 2
        if puts[kb] is not None:
            puts[kb].wait()
        gets[kb] = pltpu.async_copy(cb_hbm.at[sc_v[k]], row_v[kb], sem[kb])
        gets[kb].wait()
        puts[kb] = pltpu.async_copy(
            row_v[kb], cg_hbm.at[pl.ds(base + k * CHUNK, CHUNK), :], sem[kb])
    for p in puts:
        if p is not None:
            p.wait()


_BR = 512
_NBLK = B // _BR


def _tc_body(wr_ref, x_ref, cg_ref, sc_ref, w_ref, b_ref, cb_ref, g_ref,
             o_ref, info_ref, xn_ref, s_acc, n_acc, nrm_acc):
    i = pl.program_id(0)

    @pl.when(i == 0)
    def _():
        s_acc[...] = jnp.zeros_like(s_acc)
        n_acc[...] = jnp.zeros_like(n_acc)
        nrm_acc[...] = jnp.zeros_like(nrm_acc)

    x = x_ref[...]
    y = x + wr_ref[...] * cg_ref[...]
    o_ref[...] = (jnp.dot(y, w_ref[...], preferred_element_type=jnp.float32)
                  + 2.0 * b_ref[...])

    rs = jnp.sum(x * x, axis=1, keepdims=True)
    nrm_acc[...] += jnp.sum(jnp.sqrt(rs), keepdims=True)

    mids = lax.broadcasted_iota(jnp.int32, (_BR, M), 1)
    e = (sc_ref[...] == mids).astype(jnp.float32)
    s_acc[...] += lax.dot_general(e, x, (((0,), (0,)), ((), ())),
                                  preferred_element_type=jnp.float32)
    n_acc[...] += jnp.sum(e, axis=0, keepdims=True)

    @pl.when(i == _NBLK - 1)
    def _():
        wr = wr_ref[...]
        z = wr * cb_ref[...] + s_acc[...]
        zw = jnp.dot(z, w_ref[...], preferred_element_type=jnp.float32)
        g = g_ref[...]
        part1 = jnp.sum(zw * g, keepdims=True)
        gb = jnp.sum(g * b_ref[...], axis=1, keepdims=True)
        part2 = jnp.dot(1.0 + n_acc[...], gb,
                        preferred_element_type=jnp.float32)
        info_ref[...] = wr * (part1 + part2)
        xn_ref[...] = nrm_acc[...] / B


_tc_main = pl.pallas_call(
    _tc_body,
    grid=(_NBLK,),
    in_specs=[
        pl.BlockSpec((1, 1), lambda i: (0, 0)),
        pl.BlockSpec((_BR, D), lambda i: (i, 0)),
        pl.BlockSpec((_BR, D), lambda i: (i, 0)),
        pl.BlockSpec((_BR, 1), lambda i: (i, 0)),
        pl.BlockSpec((D, D), lambda i: (0, 0)),
        pl.BlockSpec((1, D), lambda i: (0, 0)),
        pl.BlockSpec((M, D), lambda i: (0, 0)),
        pl.BlockSpec((M, D), lambda i: (0, 0)),
    ],
    out_specs=[
        pl.BlockSpec((_BR, D), lambda i: (i, 0)),
        pl.BlockSpec((1, 1), lambda i: (0, 0)),
        pl.BlockSpec((1, 1), lambda i: (0, 0)),
    ],
    out_shape=[
        jax.ShapeDtypeStruct((B, D), jnp.float32),
        jax.ShapeDtypeStruct((1, 1), jnp.float32),
        jax.ShapeDtypeStruct((1, 1), jnp.float32),
    ],
    scratch_shapes=[
        pltpu.VMEM((M, D), jnp.float32),
        pltpu.VMEM((1, M), jnp.float32),
        pltpu.VMEM((1, 1), jnp.float32),
    ],
)


def kernel(X_B, batch_A, c_indices, codebook, ema_grad, W, b,
           warm_up_rate=1.0, unlabeled=0):
    src = batch_A[0]
    wr = jnp.asarray(warm_up_rate, jnp.float32).reshape(1, 1)
    b_row = jnp.asarray(b, jnp.float32).reshape(1, D)

    npad = (-c_indices.shape[0]) % 128
    cind = jnp.concatenate([c_indices, jnp.zeros((npad,), jnp.int32)])
    cg, sc = _sc_gather(src.reshape(B // CHUNK, CHUNK), cind, codebook)

    out, info, xn = _tc_main(wr, X_B, cg, sc.reshape(B, 1), W, b_row,
                             codebook, ema_grad)

    zero = jnp.float32(0.0)
    return (out, zero, xn[0, 0], zero, zero, info[0, 0], X_B)
